# baseline (speedup 1.0000x reference)
import functools

import jax
import jax.numpy as jnp
from jax import lax
from jax.experimental import pallas as pl
from jax.experimental.pallas import tpu as pltpu
from jax.experimental.pallas import tpu_sc as plsc

_INFO = plsc.get_sparse_core_info()
_NC = _INFO.num_cores
_NS = _INFO.num_subcores
_NW = _NC * _NS


def _emb_gather(idx_flat, tab_lin, K, N, D):
    NCB = N // 128
    NBLK = K * NCB
    BPW = NBLK // _NW
    mesh = plsc.VectorSubcoreMesh(core_axis_name="c", subcore_axis_name="s")

    @functools.partial(
        pl.kernel,
        mesh=mesh,
        out_type=jax.ShapeDtypeStruct((K, D, N), jnp.float32),
        scratch_types=[
            pltpu.VMEM((128,), jnp.int32),
            pltpu.VMEM((128,), jnp.int32),
            pltpu.VMEM((128, 128), jnp.float32),
            pltpu.VMEM((D, 128), jnp.float32),
            pltpu.SemaphoreType.DMA,
        ],
        compiler_params=pltpu.CompilerParams(needs_layout_passes=False),
    )
    def emb(idx_hbm, tab_hbm, out_hbm, idxv, bigv, rows, outv, gsem):
        wid = lax.axis_index("s") * _NC + lax.axis_index("c")
        lane = lax.iota(jnp.int32, 16)

        def body(i, carry):
            blk = wid * BPW + i
            k = blk // NCB
            c = blk % NCB
            pltpu.sync_copy(idx_hbm.at[pl.ds(blk * 128, 128)], idxv)
            for jg in range(8):
                v = idxv[pl.ds(jg * 16, 16)]
                bigv[pl.ds(jg * 16, 16)] = v >> 2
            pltpu.async_copy(tab_hbm.at[bigv], rows, gsem).wait()
            for jg in range(8):
                v = idxv[pl.ds(jg * 16, 16)]
                sub = (v & 3) * 32
                j16 = lane + jg * 16
                for d in range(D):
                    outv[d, pl.ds(jg * 16, 16)] = plsc.load_gather(
                        rows, [j16, sub + d]
                    )
            pltpu.sync_copy(outv, out_hbm.at[k, pl.ds(0, D), pl.ds(c * 128, 128)])
            return carry

        lax.fori_loop(0, BPW, body, 0)

    return emb(idx_flat, tab_lin)


def kernel(input, table):
    n, k = input.shape
    D = table.shape[1]
    idx_flat = input.astype(jnp.int32).T.reshape(-1)
    tab_lin = table.reshape(table.shape[0] // 4, 4 * D)
    out = _emb_gather(idx_flat, tab_lin, k, n, D)
    return out.transpose(2, 0, 1)


# speedup vs baseline: 1.2090x; 1.2090x over previous
import functools

import jax
import jax.numpy as jnp
from jax import lax
from jax.experimental import pallas as pl
from jax.experimental.pallas import tpu as pltpu
from jax.experimental.pallas import tpu_sc as plsc

_INFO = plsc.get_sparse_core_info()
_NC = _INFO.num_cores
_NS = _INFO.num_subcores
_NW = _NC * _NS


def _emb_gather(idx_flat, tab_lin, K, N, D):
    NCB = N // 128
    NBLK = K * NCB
    BPW = NBLK // _NW
    mesh = plsc.VectorSubcoreMesh(core_axis_name="c", subcore_axis_name="s")

    IPW = BPW * 128

    @functools.partial(
        pl.kernel,
        mesh=mesh,
        out_type=jax.ShapeDtypeStruct((K, D, N), jnp.float32),
        scratch_types=[
            pltpu.VMEM((IPW,), jnp.int32),
            pltpu.VMEM((IPW,), jnp.int32),
            pltpu.VMEM((2, 128, 128), jnp.float32),
            pltpu.VMEM((2, D, 128), jnp.float32),
            pltpu.SemaphoreType.DMA((2,)),
            pltpu.SemaphoreType.DMA((2,)),
        ],
        compiler_params=pltpu.CompilerParams(needs_layout_passes=False),
    )
    def emb(idx_hbm, tab_hbm, out_hbm, idxv, bigv, rows, outv, gsem, osem):
        wid = lax.axis_index("s") * _NC + lax.axis_index("c")
        lane = lax.iota(jnp.int32, 16)
        pltpu.sync_copy(idx_hbm.at[pl.ds(wid * IPW, IPW)], idxv)

        def cb(i, carry):
            bigv[pl.ds(i * 16, 16)] = idxv[pl.ds(i * 16, 16)] >> 2
            return carry

        lax.fori_loop(0, IPW // 16, cb, 0)

        def gather(b, buf):
            return pltpu.make_async_copy(
                tab_hbm.at[bigv.at[pl.ds(b * 128, 128)]], rows.at[buf],
                gsem.at[buf],
            )

        def store(b, buf):
            blk = wid * BPW + b
            return pltpu.make_async_copy(
                outv.at[buf],
                out_hbm.at[blk // NCB, pl.ds(0, D),
                           pl.ds((blk % NCB) * 128, 128)],
                osem.at[buf],
            )

        gather(0, 0).start()
        gather(1, 1).start()

        def body(g, carry):
            for buf in range(2):
                b = g * 2 + buf
                gather(b, buf).wait()

                @pl.when(b >= 2)
                def _drain():
                    store(b - 2, buf).wait()

                for jg in range(8):
                    v = idxv[pl.ds(b * 128 + jg * 16, 16)]
                    sub = (v & 3) * D
                    j16 = lane + jg * 16
                    for d in range(D):
                        outv[buf, d, pl.ds(jg * 16, 16)] = plsc.load_gather(
                            rows.at[buf], [j16, sub + d]
                        )
                store(b, buf).start()

                @pl.when(b + 2 < BPW)
                def _next():
                    gather(b + 2, buf).start()

            return carry

        lax.fori_loop(0, BPW // 2, body, 0)
        store(BPW - 2, 0).wait()
        store(BPW - 1, 1).wait()

    return emb(idx_flat, tab_lin)


def kernel(input, table):
    n, k = input.shape
    D = table.shape[1]
    idx_flat = input.astype(jnp.int32).T.reshape(-1)
    tab_lin = table.reshape(table.shape[0] // 4, 4 * D)
    out = _emb_gather(idx_flat, tab_lin, k, n, D)
    return out.transpose(2, 0, 1)


# speedup vs baseline: 1.4551x; 1.2035x over previous
import functools

import jax
import jax.numpy as jnp
from jax import lax
from jax.experimental import pallas as pl
from jax.experimental.pallas import tpu as pltpu
from jax.experimental.pallas import tpu_sc as plsc

_INFO = plsc.get_sparse_core_info()
_NC = _INFO.num_cores
_NS = _INFO.num_subcores
_NW = _NC * _NS


def _emb_gather(idx_flat, table, K, N, D):
    NCB = N // 128
    NBLK = K * NCB
    BPW = NBLK // _NW
    IPW = BPW * 128
    mesh = plsc.VectorSubcoreMesh(core_axis_name="c", subcore_axis_name="s")

    @functools.partial(
        pl.kernel,
        mesh=mesh,
        out_type=jax.ShapeDtypeStruct((K, D, N), jnp.float32),
        scratch_types=[
            pltpu.VMEM((IPW,), jnp.int32),
            pltpu.VMEM((IPW,), jnp.int32),
            pltpu.VMEM((2, 128, 128), jnp.float32),
            pltpu.VMEM((2, D, 128), jnp.float32),
            pltpu.SemaphoreType.DMA((2,)),
            pltpu.SemaphoreType.DMA((2,)),
        ],
        compiler_params=pltpu.CompilerParams(needs_layout_passes=False),
    )
    def emb(idx_hbm, tab_hbm, out_hbm, idxv, bigv, rows, outv, gsem, osem):
        wid = lax.axis_index("s") * _NC + lax.axis_index("c")
        lane = lax.iota(jnp.int32, 16)
        pltpu.sync_copy(idx_hbm.at[pl.ds(wid * IPW, IPW)], idxv)

        def cb(i, carry):
            bigv[pl.ds(i * 16, 16)] = idxv[pl.ds(i * 16, 16)] >> 2
            return carry

        lax.fori_loop(0, IPW // 16, cb, 0)

        def gather(b, buf):
            return pltpu.make_async_copy(
                tab_hbm.at[bigv.at[pl.ds(b * 128, 128)]], rows.at[buf],
                gsem.at[buf],
            )

        def store(b, buf):
            blk = wid * BPW + b
            return pltpu.make_async_copy(
                outv.at[buf],
                out_hbm.at[blk // NCB, pl.ds(0, D),
                           pl.ds((blk % NCB) * 128, 128)],
                osem.at[buf],
            )

        gather(0, 0).start()
        gather(1, 1).start()

        def body(g, carry):
            for buf in range(2):
                b = g * 2 + buf
                gather(b, buf).wait()

                @pl.when(b >= 2)
                def _drain():
                    store(b - 2, buf).wait()

                for jg in range(8):
                    j16 = lane + jg * 16
                    v = idxv[pl.ds(b * 128 + jg * 16, 16)]
                    sub = (v & 3) * D
                    for t in range(D):
                        dvec = (lane + t) & (D - 1)
                        vals = plsc.load_gather(rows.at[buf], [j16, sub + dvec])
                        plsc.store_scatter(outv.at[buf], [dvec, j16], vals)
                store(b, buf).start()

                @pl.when(b + 2 < BPW)
                def _next():
                    gather(b + 2, buf).start()

            return carry

        lax.fori_loop(0, BPW // 2, body, 0)
        store(BPW - 2, 0).wait()
        store(BPW - 1, 1).wait()

    return emb(idx_flat, table)


def kernel(input, table):
    n, k = input.shape
    D = table.shape[1]
    idx_flat = input.astype(jnp.int32).T.reshape(-1)
    tab_lin = table.reshape(table.shape[0] // 4, 4 * D)
    out = _emb_gather(idx_flat, tab_lin, k, n, D)
    return out.transpose(2, 0, 1)
